# baseline (speedup 1.0000x reference)
import functools

import jax
import jax.numpy as jnp
from jax import lax
from jax.experimental import pallas as pl
from jax.experimental.pallas import tpu as pltpu
from jax.experimental.pallas import tpu_sc as plsc

_LANE = 16
_BT = 128
_HT = 8
_HHALF = 4
_UNIT = _HHALF * _BT


_DPC = 160


def _depad_sc(table):
    V, D = table.shape
    M = _DPC * D // 128
    n_chunks = V // _DPC
    mesh = plsc.VectorSubcoreMesh(core_axis_name="c", subcore_axis_name="s")
    info = plsc.get_sparse_core_info()
    num_cores = info.num_cores
    nw = num_cores * info.num_subcores

    @functools.partial(
        pl.kernel,
        out_type=jax.ShapeDtypeStruct((V * D // 128, 128), jnp.float32),
        mesh=mesh,
        scratch_types=[
            pltpu.VMEM((_DPC, D), jnp.float32),
            pltpu.VMEM((M, 128), jnp.float32),
        ],
        compiler_params=pltpu.CompilerParams(use_tc_tiling_on_sc=True),
    )
    def body(in_hbm, out_hbm, buf_in, buf_out):
        w = lax.axis_index("s") * num_cores + lax.axis_index("c")
        n_k = (n_chunks - w + nw - 1) // nw

        @pl.loop(0, n_k)
        def _(k):
            j = w + nw * k
            pltpu.sync_copy(in_hbm.at[pl.ds(j * _DPC, _DPC), :], buf_in)

            @pl.loop(0, M)
            def _(m):
                for kk in range(128 // D):
                    for h in range(D // _LANE):
                        buf_out[m, pl.ds(kk * D + h * _LANE, _LANE)] = (
                            buf_in[(128 // D) * m + kk, pl.ds(h * _LANE, _LANE)]
                        )

            pltpu.sync_copy(buf_out, out_hbm.at[pl.ds(j * M, M), :])

    return body(table)


def _gather_sc(idx5, table, D, TR, TC):
    H = TR * _HT
    G = D // _HT
    n_units = TR * 2
    assert n_units >= 6 and n_units % 2 == 0

    mesh = plsc.VectorSubcoreMesh(core_axis_name="c", subcore_axis_name="s")
    info = plsc.get_sparse_core_info()
    num_cores = info.num_cores

    @functools.partial(
        pl.kernel,
        out_type=jax.ShapeDtypeStruct((H, G, TC, _HT, _BT), jnp.float32),
        mesh=mesh,
        scratch_types=[
            [pltpu.VMEM((_UNIT,), jnp.int32) for _ in range(2)],
            [pltpu.VMEM((_UNIT, D), jnp.float32) for _ in range(2)],
            [pltpu.VMEM((_HHALF, G, _HT, _BT + 1), jnp.float32) for _ in range(2)],
            [pltpu.SemaphoreType.DMA for _ in range(2)],
            [pltpu.SemaphoreType.DMA for _ in range(2)],
            [pltpu.SemaphoreType.DMA for _ in range(2)],
        ],
        compiler_params=pltpu.CompilerParams(
            use_tc_tiling_on_sc=False, needs_layout_passes=False
        ),
    )
    def body(idx_hbm, table_hbm, z_hbm, idx_v, rows_v, zt_v, isem, gsem, wsem):
        w = lax.axis_index("s") * num_cores + lax.axis_index("c")
        dlane = lax.iota(jnp.int32, _LANE)
        gv = [(dlane + d0) // _HT for d0 in range(0, D, _LANE)]
        rv = [(dlane + d0) % _HT for d0 in range(0, D, _LANE)]

        def idx_load(u, b):
            R = u // 2
            half = u % 2
            pltpu.make_async_copy(
                idx_hbm.at[R, w, pl.ds(half * _UNIT, _UNIT)],
                idx_v[b], isem[b],
            ).start()

        def idx_wait(b):
            pltpu.make_async_copy(
                idx_hbm.at[0, w, pl.ds(0, _UNIT)], idx_v[b], isem[b]
            ).wait()

        def gather_start(b):
            pltpu.make_async_copy(
                table_hbm.at[idx_v[b]], rows_v[b], gsem[b]
            ).start()

        def gather_wait(b):
            pltpu.make_async_copy(
                table_hbm.at[idx_v[b]], rows_v[b], gsem[b]
            ).wait()

        def transpose(b):
            @pl.loop(0, _UNIT, unroll=4)
            def _(q):
                r_ = q // _BT
                c = q % _BT
                rf = jnp.full((_LANE,), r_, jnp.int32)
                cf = jnp.full((_LANE,), c, jnp.int32)
                for k in range(D // _LANE):
                    vec = rows_v[b][q, pl.ds(k * _LANE, _LANE)]
                    plsc.store_scatter(zt_v[b], [rf, gv[k], rv[k], cf], vec)

        def write_start(u, b):
            R = u // 2
            half = u % 2
            for r_ in range(_HHALF):
                pltpu.make_async_copy(
                    zt_v[b].at[r_, :, :, pl.ds(0, _BT)],
                    z_hbm.at[R * _HT + half * _HHALF + r_, :, w],
                    wsem[b],
                ).start()

        def write_drain(b):
            for r_ in range(_HHALF):
                pltpu.make_async_copy(
                    zt_v[b].at[r_, :, :, pl.ds(0, _BT)],
                    z_hbm.at[0, :, w], wsem[b],
                ).wait()

        def step(u, b, drain, load_next):
            p = 1 - b
            idx_wait(b)
            gather_start(b)
            gather_wait(p)
            if load_next:
                idx_load(u + 1, p)
            if drain:
                write_drain(p)
            transpose(p)
            write_start(u - 1, p)

        pltpu.sync_copy(idx_hbm.at[0, w, pl.ds(0, _UNIT)], idx_v[0])
        gather_start(0)
        idx_load(1, 1)
        step(1, 1, drain=False, load_next=True)
        step(2, 0, drain=False, load_next=True)
        step(3, 1, drain=True, load_next=True)

        @pl.loop(4, n_units - 2, step=2)
        def _(u0):
            step(u0, 0, drain=True, load_next=True)
            step(u0 + 1, 1, drain=True, load_next=True)

        step(n_units - 2, 0, drain=True, load_next=True)
        step(n_units - 1, 1, drain=True, load_next=False)
        gather_wait(1)
        write_drain(1)
        transpose(1)
        write_start(n_units - 1, 1)
        write_drain(0)
        write_drain(1)

    return body(idx5, table)


def kernel(batch_data, table):
    B, H = batch_data.shape
    V, D = table.shape
    TR = H // _HT
    TC = B // _BT
    idx5 = (
        batch_data.T.reshape(TR, _HT, TC, _BT)
        .transpose(0, 2, 1, 3)
        .reshape(TR, TC, _HT * _BT)
    )
    t_lin = _depad_sc(table).reshape(V, D)
    z = _gather_sc(idx5, t_lin, D, TR, TC)
    out = z.transpose(2, 4, 0, 1, 3).reshape(B, H, D)
    return out


# speedup vs baseline: 1.1387x; 1.1387x over previous
import functools

import jax
import jax.numpy as jnp
from jax import lax
from jax.experimental import pallas as pl
from jax.experimental.pallas import tpu as pltpu
from jax.experimental.pallas import tpu_sc as plsc

_LANE = 16
_BT = 128
_HT = 8
_HHALF = 4
_UNIT = _HHALF * _BT


_DPC = 160


def _depad_sc(table):
    V, D = table.shape
    N = _DPC // _HT
    M = _DPC * D // 128
    n_chunks = V // _DPC
    mesh = plsc.VectorSubcoreMesh(core_axis_name="c", subcore_axis_name="s")
    info = plsc.get_sparse_core_info()
    num_cores = info.num_cores
    nw = num_cores * info.num_subcores

    @functools.partial(
        pl.kernel,
        out_type=jax.ShapeDtypeStruct((V * D // 128, 128), jnp.float32),
        mesh=mesh,
        scratch_types=[
            pltpu.VMEM((N, _HT, D), jnp.float32),
            pltpu.VMEM((M, 128), jnp.float32),
        ],
        compiler_params=pltpu.CompilerParams(use_tc_tiling_on_sc=True),
    )
    def body(in_hbm, out_hbm, buf_in, buf_out):
        w = lax.axis_index("s") * num_cores + lax.axis_index("c")
        n_k = (n_chunks - w + nw - 1) // nw

        @pl.loop(0, n_k)
        def _(k):
            j = w + nw * k
            pltpu.sync_copy(in_hbm.at[pl.ds(j * N, N), :, :], buf_in)

            @pl.loop(0, N)
            def _(n):
                for r in range(_HT):
                    t = n * _HT + r
                    for h in range(D // _LANE):
                        buf_out[
                            t // (128 // D),
                            pl.ds((t % (128 // D)) * D + h * _LANE, _LANE),
                        ] = buf_in[n, r, pl.ds(h * _LANE, _LANE)]

            pltpu.sync_copy(buf_out, out_hbm.at[pl.ds(j * M, M), :])

    return body(table.reshape(V // _HT, _HT, D))


def _gather_sc(idx5, table, D, TR, TC):
    H = TR * _HT
    G = D // _HT
    n_units = TR * 2
    assert n_units >= 6 and n_units % 2 == 0

    mesh = plsc.VectorSubcoreMesh(core_axis_name="c", subcore_axis_name="s")
    info = plsc.get_sparse_core_info()
    num_cores = info.num_cores

    @functools.partial(
        pl.kernel,
        out_type=jax.ShapeDtypeStruct((H, G, TC, _HT, _BT), jnp.float32),
        mesh=mesh,
        scratch_types=[
            [pltpu.VMEM((_UNIT,), jnp.int32) for _ in range(2)],
            [pltpu.VMEM((_UNIT, D), jnp.float32) for _ in range(2)],
            [pltpu.VMEM((_HHALF, G, _HT, _BT + 1), jnp.float32) for _ in range(2)],
            [pltpu.SemaphoreType.DMA for _ in range(2)],
            [pltpu.SemaphoreType.DMA for _ in range(2)],
            [pltpu.SemaphoreType.DMA for _ in range(2)],
        ],
        compiler_params=pltpu.CompilerParams(
            use_tc_tiling_on_sc=False, needs_layout_passes=False
        ),
    )
    def body(idx_hbm, table_hbm, z_hbm, idx_v, rows_v, zt_v, isem, gsem, wsem):
        w = lax.axis_index("s") * num_cores + lax.axis_index("c")
        dlane = lax.iota(jnp.int32, _LANE)
        gv = [(dlane + d0) // _HT for d0 in range(0, D, _LANE)]
        rv = [(dlane + d0) % _HT for d0 in range(0, D, _LANE)]

        def idx_load(u, b):
            R = u // 2
            half = u % 2
            pltpu.make_async_copy(
                idx_hbm.at[R, w, pl.ds(half * _UNIT, _UNIT)],
                idx_v[b], isem[b],
            ).start()

        def idx_wait(b):
            pltpu.make_async_copy(
                idx_hbm.at[0, w, pl.ds(0, _UNIT)], idx_v[b], isem[b]
            ).wait()

        def gather_start(b):
            pltpu.make_async_copy(
                table_hbm.at[idx_v[b]], rows_v[b], gsem[b]
            ).start()

        def gather_wait(b):
            pltpu.make_async_copy(
                table_hbm.at[idx_v[b]], rows_v[b], gsem[b]
            ).wait()

        def transpose(b):
            @pl.loop(0, _UNIT, unroll=4)
            def _(q):
                r_ = q // _BT
                c = q % _BT
                rf = jnp.full((_LANE,), r_, jnp.int32)
                cf = jnp.full((_LANE,), c, jnp.int32)
                for k in range(D // _LANE):
                    vec = rows_v[b][q, pl.ds(k * _LANE, _LANE)]
                    plsc.store_scatter(zt_v[b], [rf, gv[k], rv[k], cf], vec)

        def write_start(u, b):
            R = u // 2
            half = u % 2
            for r_ in range(_HHALF):
                pltpu.make_async_copy(
                    zt_v[b].at[r_, :, :, pl.ds(0, _BT)],
                    z_hbm.at[R * _HT + half * _HHALF + r_, :, w],
                    wsem[b],
                ).start()

        def write_drain(b):
            for r_ in range(_HHALF):
                pltpu.make_async_copy(
                    zt_v[b].at[r_, :, :, pl.ds(0, _BT)],
                    z_hbm.at[0, :, w], wsem[b],
                ).wait()

        def step(u, b, drain, load_next):
            p = 1 - b
            idx_wait(b)
            gather_start(b)
            gather_wait(p)
            if load_next:
                idx_load(u + 1, p)
            if drain:
                write_drain(p)
            transpose(p)
            write_start(u - 1, p)

        pltpu.sync_copy(idx_hbm.at[0, w, pl.ds(0, _UNIT)], idx_v[0])
        gather_start(0)
        idx_load(1, 1)
        step(1, 1, drain=False, load_next=True)
        step(2, 0, drain=False, load_next=True)
        step(3, 1, drain=True, load_next=True)

        @pl.loop(4, n_units - 2, step=2)
        def _(u0):
            step(u0, 0, drain=True, load_next=True)
            step(u0 + 1, 1, drain=True, load_next=True)

        step(n_units - 2, 0, drain=True, load_next=True)
        step(n_units - 1, 1, drain=True, load_next=False)
        gather_wait(1)
        write_drain(1)
        transpose(1)
        write_start(n_units - 1, 1)
        write_drain(0)
        write_drain(1)

    return body(idx5, table)


def kernel(batch_data, table):
    B, H = batch_data.shape
    V, D = table.shape
    TR = H // _HT
    TC = B // _BT
    idx5 = (
        batch_data.T.reshape(TR, _HT, TC, _BT)
        .transpose(0, 2, 1, 3)
        .reshape(TR, TC, _HT * _BT)
    )
    t_lin = _depad_sc(table).reshape(V, D)
    z = _gather_sc(idx5, t_lin, D, TR, TC)
    out = z.transpose(2, 4, 0, 1, 3).reshape(B, H, D)
    return out


# speedup vs baseline: 1.6743x; 1.4703x over previous
import functools

import jax
import jax.numpy as jnp
from jax import lax
from jax.experimental import pallas as pl
from jax.experimental.pallas import tpu as pltpu
from jax.experimental.pallas import tpu_sc as plsc

_LANE = 16
_BT = 128
_HT = 8
_HHALF = 4
_UNIT = _HHALF * _BT


_DPC = 320


def _depad_sc(table):
    V, D = table.shape
    N = _DPC // _HT
    M = _DPC * D // 128
    n_chunks = V // _DPC
    mesh = plsc.VectorSubcoreMesh(core_axis_name="c", subcore_axis_name="s")
    info = plsc.get_sparse_core_info()
    num_cores = info.num_cores
    nw = num_cores * info.num_subcores

    @functools.partial(
        pl.kernel,
        out_type=jax.ShapeDtypeStruct((V * D // 128, 128), jnp.float32),
        mesh=mesh,
        scratch_types=[
            pltpu.VMEM((2, N, _HT, D), jnp.float32),
            pltpu.VMEM((2, M, 128), jnp.float32),
            pltpu.SemaphoreType.DMA((2,)),
            pltpu.SemaphoreType.DMA((2,)),
        ],
        compiler_params=pltpu.CompilerParams(use_tc_tiling_on_sc=True),
    )
    def body(in_hbm, out_hbm, buf_in, buf_out, rsem, wsem):
        w = lax.axis_index("s") * num_cores + lax.axis_index("c")
        n_k = (n_chunks - w + nw - 1) // nw

        def read_start(k, b):
            j = w + nw * k
            pltpu.make_async_copy(
                in_hbm.at[pl.ds(j * N, N), :, :], buf_in.at[b], rsem.at[b]
            ).start()

        def read_wait(b):
            pltpu.make_async_copy(
                in_hbm.at[pl.ds(0, N), :, :], buf_in.at[b], rsem.at[b]
            ).wait()

        def write_start(k, b):
            j = w + nw * k
            pltpu.make_async_copy(
                buf_out.at[b], out_hbm.at[pl.ds(j * M, M), :], wsem.at[b]
            ).start()

        def write_wait(b):
            pltpu.make_async_copy(
                buf_out.at[b], out_hbm.at[pl.ds(0, M), :], wsem.at[b]
            ).wait()

        read_start(0, 0)
        read_start(1, 1)

        @pl.loop(0, n_k)
        def _(k):
            b = k % 2
            read_wait(b)

            @pl.when(k >= 2)
            def _():
                write_wait(b)

            @pl.loop(0, N)
            def _(n):
                for r in range(_HT):
                    t = n * _HT + r
                    for h in range(D // _LANE):
                        buf_out[
                            b,
                            t // (128 // D),
                            pl.ds((t % (128 // D)) * D + h * _LANE, _LANE),
                        ] = buf_in[b, n, r, pl.ds(h * _LANE, _LANE)]

            write_start(k, b)

            @pl.when(k + 2 < n_k)
            def _():
                read_start(k + 2, b)

        write_wait(0)
        write_wait(1)

    return body(table.reshape(V // _HT, _HT, D))


def _gather_sc(idx5, table, D, TR, TC):
    H = TR * _HT
    G = D // _HT
    n_units = TR * 2
    assert n_units >= 6 and n_units % 2 == 0

    mesh = plsc.VectorSubcoreMesh(core_axis_name="c", subcore_axis_name="s")
    info = plsc.get_sparse_core_info()
    num_cores = info.num_cores

    @functools.partial(
        pl.kernel,
        out_type=jax.ShapeDtypeStruct((H, G, TC, _HT, _BT), jnp.float32),
        mesh=mesh,
        scratch_types=[
            [pltpu.VMEM((_UNIT,), jnp.int32) for _ in range(2)],
            [pltpu.VMEM((_UNIT, D), jnp.float32) for _ in range(2)],
            [pltpu.VMEM((_HHALF, G, _HT, _BT + 1), jnp.float32) for _ in range(2)],
            [pltpu.SemaphoreType.DMA for _ in range(2)],
            [pltpu.SemaphoreType.DMA for _ in range(2)],
            [pltpu.SemaphoreType.DMA for _ in range(2)],
        ],
        compiler_params=pltpu.CompilerParams(
            use_tc_tiling_on_sc=False, needs_layout_passes=False
        ),
    )
    def body(idx_hbm, table_hbm, z_hbm, idx_v, rows_v, zt_v, isem, gsem, wsem):
        w = lax.axis_index("s") * num_cores + lax.axis_index("c")
        dlane = lax.iota(jnp.int32, _LANE)
        gv = [(dlane + d0) // _HT for d0 in range(0, D, _LANE)]
        rv = [(dlane + d0) % _HT for d0 in range(0, D, _LANE)]

        def idx_load(u, b):
            R = u // 2
            half = u % 2
            pltpu.make_async_copy(
                idx_hbm.at[R, w, pl.ds(half * _UNIT, _UNIT)],
                idx_v[b], isem[b],
            ).start()

        def idx_wait(b):
            pltpu.make_async_copy(
                idx_hbm.at[0, w, pl.ds(0, _UNIT)], idx_v[b], isem[b]
            ).wait()

        def gather_start(b):
            pltpu.make_async_copy(
                table_hbm.at[idx_v[b]], rows_v[b], gsem[b]
            ).start()

        def gather_wait(b):
            pltpu.make_async_copy(
                table_hbm.at[idx_v[b]], rows_v[b], gsem[b]
            ).wait()

        def transpose(b):
            @pl.loop(0, _UNIT, unroll=4)
            def _(q):
                r_ = q // _BT
                c = q % _BT
                rf = jnp.full((_LANE,), r_, jnp.int32)
                cf = jnp.full((_LANE,), c, jnp.int32)
                for k in range(D // _LANE):
                    vec = rows_v[b][q, pl.ds(k * _LANE, _LANE)]
                    plsc.store_scatter(zt_v[b], [rf, gv[k], rv[k], cf], vec)

        def write_start(u, b):
            R = u // 2
            half = u % 2
            for r_ in range(_HHALF):
                pltpu.make_async_copy(
                    zt_v[b].at[r_, :, :, pl.ds(0, _BT)],
                    z_hbm.at[R * _HT + half * _HHALF + r_, :, w],
                    wsem[b],
                ).start()

        def write_drain(b):
            for r_ in range(_HHALF):
                pltpu.make_async_copy(
                    zt_v[b].at[r_, :, :, pl.ds(0, _BT)],
                    z_hbm.at[0, :, w], wsem[b],
                ).wait()

        def step(u, b, drain, load_next):
            p = 1 - b
            idx_wait(b)
            gather_start(b)
            gather_wait(p)
            if load_next:
                idx_load(u + 1, p)
            if drain:
                write_drain(p)
            transpose(p)
            write_start(u - 1, p)

        pltpu.sync_copy(idx_hbm.at[0, w, pl.ds(0, _UNIT)], idx_v[0])
        gather_start(0)
        idx_load(1, 1)
        step(1, 1, drain=False, load_next=True)
        step(2, 0, drain=False, load_next=True)
        step(3, 1, drain=True, load_next=True)

        @pl.loop(4, n_units - 2, step=2)
        def _(u0):
            step(u0, 0, drain=True, load_next=True)
            step(u0 + 1, 1, drain=True, load_next=True)

        step(n_units - 2, 0, drain=True, load_next=True)
        step(n_units - 1, 1, drain=True, load_next=False)
        gather_wait(1)
        write_drain(1)
        transpose(1)
        write_start(n_units - 1, 1)
        write_drain(0)
        write_drain(1)

    return body(idx5, table)


def kernel(batch_data, table):
    B, H = batch_data.shape
    V, D = table.shape
    TR = H // _HT
    TC = B // _BT
    idx5 = (
        batch_data.T.reshape(TR, _HT, TC, _BT)
        .transpose(0, 2, 1, 3)
        .reshape(TR, TC, _HT * _BT)
    )
    t_lin = _depad_sc(table).reshape(V, D)
    z = _gather_sc(idx5, t_lin, D, TR, TC)
    out = z.transpose(2, 4, 0, 1, 3).reshape(B, H, D)
    return out


# speedup vs baseline: 1.7304x; 1.0335x over previous
import functools

import jax
import jax.numpy as jnp
from jax import lax
from jax.experimental import pallas as pl
from jax.experimental.pallas import tpu as pltpu
from jax.experimental.pallas import tpu_sc as plsc

_LANE = 16
_BT = 128
_HT = 8
_HHALF = 4
_UNIT = _HHALF * _BT


_DPC = 320


def _depad_sc(table):
    V, D = table.shape
    N = _DPC // _HT
    M = _DPC * D // 128
    n_chunks = V // _DPC
    mesh = plsc.VectorSubcoreMesh(core_axis_name="c", subcore_axis_name="s")
    info = plsc.get_sparse_core_info()
    num_cores = info.num_cores
    nw = num_cores * info.num_subcores

    @functools.partial(
        pl.kernel,
        out_type=jax.ShapeDtypeStruct((V * D // 128, 128), jnp.float32),
        mesh=mesh,
        scratch_types=[
            pltpu.VMEM((2, N, _HT, D), jnp.float32),
            pltpu.VMEM((2, M, 128), jnp.float32),
            pltpu.SemaphoreType.DMA((2,)),
            pltpu.SemaphoreType.DMA((2,)),
        ],
        compiler_params=pltpu.CompilerParams(use_tc_tiling_on_sc=True),
    )
    def body(in_hbm, out_hbm, buf_in, buf_out, rsem, wsem):
        w = lax.axis_index("s") * num_cores + lax.axis_index("c")
        n_k = (n_chunks - w + nw - 1) // nw

        def read_start(k, b):
            j = w + nw * k
            pltpu.make_async_copy(
                in_hbm.at[pl.ds(j * N, N), :, :], buf_in.at[b], rsem.at[b]
            ).start()

        def read_wait(b):
            pltpu.make_async_copy(
                in_hbm.at[pl.ds(0, N), :, :], buf_in.at[b], rsem.at[b]
            ).wait()

        def write_start(k, b):
            j = w + nw * k
            pltpu.make_async_copy(
                buf_out.at[b], out_hbm.at[pl.ds(j * M, M), :], wsem.at[b]
            ).start()

        def write_wait(b):
            pltpu.make_async_copy(
                buf_out.at[b], out_hbm.at[pl.ds(0, M), :], wsem.at[b]
            ).wait()

        read_start(0, 0)
        read_start(1, 1)

        @pl.loop(0, n_k)
        def _(k):
            b = k % 2
            read_wait(b)

            @pl.when(k >= 2)
            def _():
                write_wait(b)

            @pl.loop(0, N, unroll=2)
            def _(n):
                m0 = (_HT // (128 // D)) * n
                for r in range(_HT):
                    for h in range(D // _LANE):
                        buf_out[
                            b,
                            m0 + r // (128 // D),
                            pl.ds((r % (128 // D)) * D + h * _LANE, _LANE),
                        ] = buf_in[b, n, r, pl.ds(h * _LANE, _LANE)]

            write_start(k, b)

            @pl.when(k + 2 < n_k)
            def _():
                read_start(k + 2, b)

        write_wait(0)
        write_wait(1)

    return body(table.reshape(V // _HT, _HT, D))


def _gather_sc(idx5, table, D, TR, TC):
    H = TR * _HT
    G = D // _HT
    n_units = TR * 2
    assert n_units >= 6 and n_units % 2 == 0

    mesh = plsc.VectorSubcoreMesh(core_axis_name="c", subcore_axis_name="s")
    info = plsc.get_sparse_core_info()
    num_cores = info.num_cores

    @functools.partial(
        pl.kernel,
        out_type=jax.ShapeDtypeStruct((H, G, TC, _HT, _BT), jnp.float32),
        mesh=mesh,
        scratch_types=[
            [pltpu.VMEM((_UNIT,), jnp.int32) for _ in range(2)],
            [pltpu.VMEM((_UNIT, D), jnp.float32) for _ in range(2)],
            [pltpu.VMEM((_HHALF, G, _HT, _BT + 1), jnp.float32) for _ in range(2)],
            [pltpu.SemaphoreType.DMA for _ in range(2)],
            [pltpu.SemaphoreType.DMA for _ in range(2)],
            [pltpu.SemaphoreType.DMA for _ in range(2)],
        ],
        compiler_params=pltpu.CompilerParams(
            use_tc_tiling_on_sc=False, needs_layout_passes=False
        ),
    )
    def body(idx_hbm, table_hbm, z_hbm, idx_v, rows_v, zt_v, isem, gsem, wsem):
        w = lax.axis_index("s") * num_cores + lax.axis_index("c")
        dlane = lax.iota(jnp.int32, _LANE)
        gv = [(dlane + d0) // _HT for d0 in range(0, D, _LANE)]
        rv = [(dlane + d0) % _HT for d0 in range(0, D, _LANE)]

        def idx_load(u, b):
            R = u // 2
            half = u % 2
            pltpu.make_async_copy(
                idx_hbm.at[R, w, pl.ds(half * _UNIT, _UNIT)],
                idx_v[b], isem[b],
            ).start()

        def idx_wait(b):
            pltpu.make_async_copy(
                idx_hbm.at[0, w, pl.ds(0, _UNIT)], idx_v[b], isem[b]
            ).wait()

        def gather_start(b):
            pltpu.make_async_copy(
                table_hbm.at[idx_v[b]], rows_v[b], gsem[b]
            ).start()

        def gather_wait(b):
            pltpu.make_async_copy(
                table_hbm.at[idx_v[b]], rows_v[b], gsem[b]
            ).wait()

        def transpose(b):
            @pl.loop(0, _UNIT, unroll=4)
            def _(q):
                r_ = q // _BT
                c = q % _BT
                rf = jnp.full((_LANE,), r_, jnp.int32)
                cf = jnp.full((_LANE,), c, jnp.int32)
                for k in range(D // _LANE):
                    vec = rows_v[b][q, pl.ds(k * _LANE, _LANE)]
                    plsc.store_scatter(zt_v[b], [rf, gv[k], rv[k], cf], vec)

        def write_start(u, b):
            R = u // 2
            half = u % 2
            for r_ in range(_HHALF):
                pltpu.make_async_copy(
                    zt_v[b].at[r_, :, :, pl.ds(0, _BT)],
                    z_hbm.at[R * _HT + half * _HHALF + r_, :, w],
                    wsem[b],
                ).start()

        def write_drain(b):
            for r_ in range(_HHALF):
                pltpu.make_async_copy(
                    zt_v[b].at[r_, :, :, pl.ds(0, _BT)],
                    z_hbm.at[0, :, w], wsem[b],
                ).wait()

        def step(u, b, drain, load_next):
            p = 1 - b
            idx_wait(b)
            gather_start(b)
            gather_wait(p)
            if load_next:
                idx_load(u + 1, p)
            if drain:
                write_drain(p)
            transpose(p)
            write_start(u - 1, p)

        pltpu.sync_copy(idx_hbm.at[0, w, pl.ds(0, _UNIT)], idx_v[0])
        gather_start(0)
        idx_load(1, 1)
        step(1, 1, drain=False, load_next=True)
        step(2, 0, drain=False, load_next=True)
        step(3, 1, drain=True, load_next=True)

        @pl.loop(4, n_units - 2, step=2)
        def _(u0):
            step(u0, 0, drain=True, load_next=True)
            step(u0 + 1, 1, drain=True, load_next=True)

        step(n_units - 2, 0, drain=True, load_next=True)
        step(n_units - 1, 1, drain=True, load_next=False)
        gather_wait(1)
        write_drain(1)
        transpose(1)
        write_start(n_units - 1, 1)
        write_drain(0)
        write_drain(1)

    return body(idx5, table)


def kernel(batch_data, table):
    B, H = batch_data.shape
    V, D = table.shape
    TR = H // _HT
    TC = B // _BT
    idx5 = (
        batch_data.T.reshape(TR, _HT, TC, _BT)
        .transpose(0, 2, 1, 3)
        .reshape(TR, TC, _HT * _BT)
    )
    t_lin = _depad_sc(table).reshape(V, D)
    z = _gather_sc(idx5, t_lin, D, TR, TC)
    out = z.transpose(2, 4, 0, 1, 3).reshape(B, H, D)
    return out
